# baseline (speedup 1.0000x reference)
import functools

import jax
import jax.numpy as jnp
from jax import lax
from jax.experimental import pallas as pl
from jax.experimental.pallas import tpu as pltpu
from jax.experimental.pallas import tpu_sc as plsc

VOCAB = 1000000
EMB = 64
HID = 128
B = 16384
L = 200
SPLIT = 128
REST = L - SPLIT
LANES = 16

NC = 2
NS = 16
NW = NC * NS
RPW = B // NW
ICH = 64
NCH = RPW // ICH
NBUF = 8


def _pool_body(x_hbm, emb_hbm, out_hbm, idx_c, rows, outc, *sems):
    c = lax.axis_index("c")
    s = lax.axis_index("s")
    wid = s * NC + c
    base = wid * RPW

    def descs(j, b):
        d0 = pltpu.make_async_copy(
            emb_hbm.at[idx_c.at[pl.ds(j * L, SPLIT)]],
            rows.at[b, pl.ds(0, SPLIT)], sems[b])
        d1 = pltpu.make_async_copy(
            emb_hbm.at[idx_c.at[pl.ds(j * L + SPLIT, REST)]],
            rows.at[b, pl.ds(SPLIT, REST)], sems[b])
        return d0, d1

    def start(j, b):
        d0, d1 = descs(j, b)
        d0.start()
        d1.start()

    def wait(j, b):
        d0, d1 = descs(j, b)
        d0.wait()
        d1.wait()

    def row_add(b, l, acc):
        new = list(acc)
        for q in range(4):
            new[q] = new[q] + rows[b, l, pl.ds(q * LANES, LANES)]
        return tuple(new)

    def process(j, b):
        def lstep(k, acc):
            for t in range(8):
                acc = row_add(b, k * 8 + t, acc)
            return acc

        zero = jnp.zeros((LANES,), jnp.float32)
        acc = lax.fori_loop(0, L // 8, lstep, (zero,) * 4)

        cnt = jnp.zeros((LANES,), jnp.int32)
        for k in range(L // LANES):
            v = idx_c[pl.ds(j * L + k * LANES, LANES)]
            cnt = cnt + plsc.all_reduce_population_count(v != 0)
        vt = idx_c[pl.ds(j * L + L - LANES, LANES)]
        tail_mask = lax.iota(jnp.int32, LANES) >= 8
        cnt = cnt + plsc.all_reduce_population_count((vt != 0) & tail_mask)
        lenf = jnp.maximum(cnt.astype(jnp.float32), 1.0)

        for q in range(4):
            outc[j, pl.ds(q * LANES, LANES)] = acc[q] / lenf

    def chunk(g, carry):
        pltpu.sync_copy(x_hbm.at[pl.ds((base + g * ICH) * L, ICH * L)], idx_c)
        for b in range(NBUF - 1):
            start(b, b)

        def grp(q, inner):
            j0 = q * NBUF
            for b in range(NBUF):
                j = j0 + b

                @pl.when(j + NBUF - 1 < ICH)
                def _():
                    start(j + NBUF - 1, (b + NBUF - 1) % NBUF)

                wait(j, b)
                process(j, b)
            return inner

        lax.fori_loop(0, ICH // NBUF, grp, carry)
        pltpu.sync_copy(outc, out_hbm.at[pl.ds(base + g * ICH, ICH)])
        return carry

    lax.fori_loop(0, NCH, chunk, 0)


@functools.partial(
    pl.kernel,
    out_type=jax.ShapeDtypeStruct((B, EMB), jnp.float32),
    mesh=plsc.VectorSubcoreMesh(core_axis_name="c", subcore_axis_name="s"),
    scratch_types=[
        pltpu.VMEM((ICH * L,), jnp.int32),
        pltpu.VMEM((NBUF, L, EMB), jnp.float32),
        pltpu.VMEM((ICH, EMB), jnp.float32),
    ] + [pltpu.SemaphoreType.DMA] * NBUF,
    compiler_params=pltpu.CompilerParams(
        use_tc_tiling_on_sc=False, needs_layout_passes=False),
)
def _pool(x_hbm, emb_hbm, out_hbm, idx_c, rows, outc, *sems):
    _pool_body(x_hbm, emb_hbm, out_hbm, idx_c, rows, outc, *sems)


def _mlp_kernel(avg_ref, w1_ref, b1_ref, w2t_ref, b2_ref, out_ref):
    h = jnp.dot(avg_ref[...], w1_ref[...],
                preferred_element_type=jnp.float32) + b1_ref[...]
    h = jnp.maximum(h, 0.0)
    out_ref[...] = jnp.sum(h * w2t_ref[...], axis=1) + b2_ref[0]


_MLP_BLK = 2048


def _mlp(avg, W1, b1, w2t, b2):
    grid = (B // _MLP_BLK,)
    return pl.pallas_call(
        _mlp_kernel,
        grid=grid,
        in_specs=[
            pl.BlockSpec((_MLP_BLK, EMB), lambda i: (i, 0)),
            pl.BlockSpec((EMB, HID), lambda i: (0, 0)),
            pl.BlockSpec((1, HID), lambda i: (0, 0)),
            pl.BlockSpec((1, HID), lambda i: (0, 0)),
            pl.BlockSpec(memory_space=pltpu.SMEM),
        ],
        out_specs=pl.BlockSpec((_MLP_BLK,), lambda i: (i,)),
        out_shape=jax.ShapeDtypeStruct((B,), jnp.float32),
    )(avg, W1, b1, w2t, b2)


def kernel(x, emb, W1, b1, W2, b2):
    avg = _pool(x.reshape(B * L), emb)
    return _mlp(avg, W1, b1.reshape(1, HID), W2.reshape(1, HID), b2)


# speedup vs baseline: 1.0208x; 1.0208x over previous
import functools

import jax
import jax.numpy as jnp
from jax import lax
from jax.experimental import pallas as pl
from jax.experimental.pallas import tpu as pltpu
from jax.experimental.pallas import tpu_sc as plsc

VOCAB = 1000000
EMB = 64
HID = 128
B = 16384
L = 200
SPLIT = 128
REST = L - SPLIT
LANES = 16

NC = 2
NS = 16
NW = NC * NS
RPW = B // NW
ICH = 128
NCH = RPW // ICH
NBUF = 4


def _pool_body(x_hbm, emb_hbm, out_hbm, idx_c, rows, outc, *sems):
    c = lax.axis_index("c")
    s = lax.axis_index("s")
    wid = s * NC + c
    base = wid * RPW

    def descs(j, b):
        d0 = pltpu.make_async_copy(
            emb_hbm.at[idx_c.at[pl.ds(j * L, SPLIT)]],
            rows.at[b, pl.ds(0, SPLIT)], sems[b])
        d1 = pltpu.make_async_copy(
            emb_hbm.at[idx_c.at[pl.ds(j * L + SPLIT, REST)]],
            rows.at[b, pl.ds(SPLIT, REST)], sems[b])
        return d0, d1

    def start(j, b):
        d0, d1 = descs(j, b)
        d0.start()
        d1.start()

    def wait(j, b):
        d0, d1 = descs(j, b)
        d0.wait()
        d1.wait()

    def row_add(b, l, acc):
        new = list(acc)
        for q in range(4):
            new[q] = new[q] + rows[b, l, pl.ds(q * LANES, LANES)]
        return tuple(new)

    def process(j, b):
        def lstep(k, acc):
            for t in range(8):
                acc = row_add(b, k * 8 + t, acc)
            return acc

        zero = jnp.zeros((LANES,), jnp.float32)
        acc = lax.fori_loop(0, L // 8, lstep, (zero,) * 4)

        cnt = jnp.zeros((LANES,), jnp.int32)
        for k in range(L // LANES):
            v = idx_c[pl.ds(j * L + k * LANES, LANES)]
            cnt = cnt + plsc.all_reduce_population_count(v != 0)
        vt = idx_c[pl.ds(j * L + L - LANES, LANES)]
        tail_mask = lax.iota(jnp.int32, LANES) >= 8
        cnt = cnt + plsc.all_reduce_population_count((vt != 0) & tail_mask)
        lenf = jnp.maximum(cnt.astype(jnp.float32), 1.0)

        for q in range(4):
            outc[j, pl.ds(q * LANES, LANES)] = acc[q] / lenf

    def chunk(g, carry):
        pltpu.sync_copy(x_hbm.at[pl.ds((base + g * ICH) * L, ICH * L)], idx_c)
        for b in range(NBUF - 1):
            start(b, b)

        def grp(q, inner):
            j0 = q * NBUF
            for b in range(NBUF):
                j = j0 + b

                @pl.when(j + NBUF - 1 < ICH)
                def _():
                    start(j + NBUF - 1, (b + NBUF - 1) % NBUF)

                wait(j, b)
                process(j, b)
            return inner

        lax.fori_loop(0, ICH // NBUF, grp, carry)
        pltpu.sync_copy(outc, out_hbm.at[pl.ds(base + g * ICH, ICH)])
        return carry

    lax.fori_loop(0, NCH, chunk, 0)


@functools.partial(
    pl.kernel,
    out_type=jax.ShapeDtypeStruct((B, EMB), jnp.float32),
    mesh=plsc.VectorSubcoreMesh(core_axis_name="c", subcore_axis_name="s"),
    scratch_types=[
        pltpu.VMEM((ICH * L,), jnp.int32),
        pltpu.VMEM((NBUF, L, EMB), jnp.float32),
        pltpu.VMEM((ICH, EMB), jnp.float32),
    ] + [pltpu.SemaphoreType.DMA] * NBUF,
    compiler_params=pltpu.CompilerParams(
        use_tc_tiling_on_sc=False, needs_layout_passes=False),
)
def _pool(x_hbm, emb_hbm, out_hbm, idx_c, rows, outc, *sems):
    _pool_body(x_hbm, emb_hbm, out_hbm, idx_c, rows, outc, *sems)


def _mlp_kernel(avg_ref, w1_ref, b1_ref, w2t_ref, b2_ref, out_ref):
    h = jnp.dot(avg_ref[...], w1_ref[...],
                preferred_element_type=jnp.float32) + b1_ref[...]
    h = jnp.maximum(h, 0.0)
    out_ref[...] = jnp.sum(h * w2t_ref[...], axis=1) + b2_ref[0]


_MLP_BLK = 2048


def _mlp(avg, W1, b1, w2t, b2):
    grid = (B // _MLP_BLK,)
    return pl.pallas_call(
        _mlp_kernel,
        grid=grid,
        in_specs=[
            pl.BlockSpec((_MLP_BLK, EMB), lambda i: (i, 0)),
            pl.BlockSpec((EMB, HID), lambda i: (0, 0)),
            pl.BlockSpec((1, HID), lambda i: (0, 0)),
            pl.BlockSpec((1, HID), lambda i: (0, 0)),
            pl.BlockSpec(memory_space=pltpu.SMEM),
        ],
        out_specs=pl.BlockSpec((_MLP_BLK,), lambda i: (i,)),
        out_shape=jax.ShapeDtypeStruct((B,), jnp.float32),
    )(avg, W1, b1, w2t, b2)


def kernel(x, emb, W1, b1, W2, b2):
    avg = _pool(x.reshape(B * L), emb)
    return _mlp(avg, W1, b1.reshape(1, HID), W2.reshape(1, HID), b2)


# speedup vs baseline: 1.0278x; 1.0068x over previous
import functools

import jax
import jax.numpy as jnp
from jax import lax
from jax.experimental import pallas as pl
from jax.experimental.pallas import tpu as pltpu
from jax.experimental.pallas import tpu_sc as plsc

VOCAB = 1000000
EMB = 64
HID = 128
B = 16384
L = 200
SPLIT = 128
REST = L - SPLIT
LANES = 16

NC = 2
NS = 16
NW = NC * NS
RPW = B // NW
ICH = 256
NCH = RPW // ICH
NBUF = 4


def _pool_body(x_hbm, emb_hbm, out_hbm, idx_c, rows, outc, *sems):
    c = lax.axis_index("c")
    s = lax.axis_index("s")
    wid = s * NC + c
    base = wid * RPW

    def descs(j, b):
        d0 = pltpu.make_async_copy(
            emb_hbm.at[idx_c.at[pl.ds(j * L, SPLIT)]],
            rows.at[b, pl.ds(0, SPLIT)], sems[b])
        d1 = pltpu.make_async_copy(
            emb_hbm.at[idx_c.at[pl.ds(j * L + SPLIT, REST)]],
            rows.at[b, pl.ds(SPLIT, REST)], sems[b])
        return d0, d1

    def start(j, b):
        d0, d1 = descs(j, b)
        d0.start()
        d1.start()

    def wait(j, b):
        d0, d1 = descs(j, b)
        d0.wait()
        d1.wait()

    def row_add(b, l, acc):
        new = list(acc)
        for q in range(4):
            new[q] = new[q] + rows[b, l, pl.ds(q * LANES, LANES)]
        return tuple(new)

    def process(j, b):
        def lstep(k, acc):
            for t in range(8):
                acc = row_add(b, k * 8 + t, acc)
            return acc

        zero = jnp.zeros((LANES,), jnp.float32)
        acc = lax.fori_loop(0, L // 8, lstep, (zero,) * 4)

        cnt = jnp.zeros((LANES,), jnp.int32)
        for k in range(L // LANES):
            v = idx_c[pl.ds(j * L + k * LANES, LANES)]
            cnt = cnt + plsc.all_reduce_population_count(v != 0)
        vt = idx_c[pl.ds(j * L + L - LANES, LANES)]
        tail_mask = lax.iota(jnp.int32, LANES) >= 8
        cnt = cnt + plsc.all_reduce_population_count((vt != 0) & tail_mask)
        lenf = jnp.maximum(cnt.astype(jnp.float32), 1.0)

        for q in range(4):
            outc[j, pl.ds(q * LANES, LANES)] = acc[q] / lenf

    def chunk(g, carry):
        pltpu.sync_copy(x_hbm.at[pl.ds((base + g * ICH) * L, ICH * L)], idx_c)
        for b in range(NBUF - 1):
            start(b, b)

        def grp(q, inner):
            j0 = q * NBUF
            for b in range(NBUF):
                j = j0 + b

                @pl.when(j + NBUF - 1 < ICH)
                def _():
                    start(j + NBUF - 1, (b + NBUF - 1) % NBUF)

                wait(j, b)
                process(j, b)
            return inner

        lax.fori_loop(0, ICH // NBUF, grp, carry)
        pltpu.sync_copy(outc, out_hbm.at[pl.ds(base + g * ICH, ICH)])
        return carry

    lax.fori_loop(0, NCH, chunk, 0)


@functools.partial(
    pl.kernel,
    out_type=jax.ShapeDtypeStruct((B, EMB), jnp.float32),
    mesh=plsc.VectorSubcoreMesh(core_axis_name="c", subcore_axis_name="s"),
    scratch_types=[
        pltpu.VMEM((ICH * L,), jnp.int32),
        pltpu.VMEM((NBUF, L, EMB), jnp.float32),
        pltpu.VMEM((ICH, EMB), jnp.float32),
    ] + [pltpu.SemaphoreType.DMA] * NBUF,
    compiler_params=pltpu.CompilerParams(
        use_tc_tiling_on_sc=False, needs_layout_passes=False),
)
def _pool(x_hbm, emb_hbm, out_hbm, idx_c, rows, outc, *sems):
    _pool_body(x_hbm, emb_hbm, out_hbm, idx_c, rows, outc, *sems)


def _mlp_kernel(avg_ref, w1_ref, b1_ref, w2t_ref, b2_ref, out_ref):
    h = jnp.dot(avg_ref[...], w1_ref[...],
                preferred_element_type=jnp.float32) + b1_ref[...]
    h = jnp.maximum(h, 0.0)
    out_ref[...] = jnp.sum(h * w2t_ref[...], axis=1) + b2_ref[0]


_MLP_BLK = 2048


def _mlp(avg, W1, b1, w2t, b2):
    grid = (B // _MLP_BLK,)
    return pl.pallas_call(
        _mlp_kernel,
        grid=grid,
        in_specs=[
            pl.BlockSpec((_MLP_BLK, EMB), lambda i: (i, 0)),
            pl.BlockSpec((EMB, HID), lambda i: (0, 0)),
            pl.BlockSpec((1, HID), lambda i: (0, 0)),
            pl.BlockSpec((1, HID), lambda i: (0, 0)),
            pl.BlockSpec(memory_space=pltpu.SMEM),
        ],
        out_specs=pl.BlockSpec((_MLP_BLK,), lambda i: (i,)),
        out_shape=jax.ShapeDtypeStruct((B,), jnp.float32),
    )(avg, W1, b1, w2t, b2)


def kernel(x, emb, W1, b1, W2, b2):
    avg = _pool(x.reshape(B * L), emb)
    return _mlp(avg, W1, b1.reshape(1, HID), W2.reshape(1, HID), b2)
